# baseline (speedup 1.0000x reference)
import functools

import jax
import jax.numpy as jnp
from jax import lax
from jax.experimental import pallas as pl
from jax.experimental.pallas import tpu as pltpu
from jax.experimental.pallas import tpu_sc as plsc

N = 10000
E = 320000
D = 128
H = 128
C = 40
HH = 64

NC = 2
NS = 16
NW = NC * NS
MB = 128
K = E // (NW * MB) + 1
KA = {128: 104, 64: 98}
KAMAX = max(KA.values())
KT = NS * 2 * K
ACCROWS = ((N + 1 + MB - 1) // MB) * MB
EPAD = KT * MB
RPT = 624
RPT_LAST = N - (NS - 1) * RPT
DEGW = 16


@functools.lru_cache(maxsize=None)
def _mesh():
    return plsc.VectorSubcoreMesh(core_axis_name="c", subcore_axis_name="s")


def _fill(ref, rows, cols, value):
    v16 = jnp.full((16,), value, jnp.float32)

    def row(i, _):
        def col(l, __):
            ref[i, pl.ds(l * 16, 16)] = v16
            return 0
        return lax.fori_loop(0, cols // 16, col, 0)

    lax.fori_loop(0, rows, row, 0)


def _zero_acc(zref, acc, sid):
    nchunks = ACCROWS // MB

    def step(t, _):
        cid = t * NS + sid

        @pl.when(cid < nchunks)
        def _():
            pltpu.sync_copy(zref, acc.at[pl.ds(cid * MB, MB)])
        return 0

    lax.fori_loop(0, (nchunks + NS - 1) // NS, step, 0)


def _copy_out(acc, out, cid, sid):
    @pl.when(sid < NS - 1)
    def _():
        pltpu.sync_copy(acc.at[pl.ds(sid * RPT, RPT)],
                        out.at[pl.ds(cid * N + sid * RPT, RPT)])

    @pl.when(sid == NS - 1)
    def _():
        pltpu.sync_copy(acc.at[pl.ds((NS - 1) * RPT, RPT_LAST)],
                        out.at[pl.ds(cid * N + (NS - 1) * RPT, RPT_LAST)])


def _agg_phase(h, zblk, out, srcb, dstb, src_v, dst_v, rows_v, acc, sem0,
               cid, sid, hp):
    ka = KA[hp]
    kb = 2 * K - ka
    plsc.subcore_barrier()
    pltpu.sync_copy(zblk, rows_v)
    _zero_acc(rows_v, acc, sid)
    plsc.subcore_barrier()

    def work(kc, start):
        pltpu.sync_copy(srcb.at[pl.ds(start, kc)], src_v.at[pl.ds(0, kc)])
        pltpu.sync_copy(dstb.at[pl.ds(start, kc)], dst_v.at[pl.ds(0, kc)])

        def step(g, _):
            pltpu.async_copy(h.at[src_v.at[g]], rows_v, sem0).wait()
            pltpu.sync_copy(rows_v, acc.at[dst_v.at[g]], add=True)
            return 0

        lax.fori_loop(0, kc, step, 0)

    @pl.when(cid == 0)
    def _():
        work(ka, sid * ka)

    @pl.when(cid == 1)
    def _():
        work(kb, NS * ka + sid * kb)

    plsc.subcore_barrier()
    _copy_out(acc, out, cid, sid)


@functools.lru_cache(maxsize=None)
def _make_deg_kernel():
    @functools.partial(
        pl.kernel,
        out_type=jax.ShapeDtypeStruct((2 * N, DEGW), jnp.float32),
        mesh=_mesh(),
        compiler_params=pltpu.CompilerParams(use_tc_tiling_on_sc=False),
        scratch_types=[
            pltpu.VMEM((K, MB), jnp.int32),
            pltpu.VMEM((MB, DEGW), jnp.float32),
            pltpu.VMEM((MB, DEGW), jnp.float32),
            pltpu.VMEM_SHARED((ACCROWS, DEGW), jnp.float32),
        ],
    )
    def deg_kernel(dstb, ones_in, out, idx_v, ones_v, zero_v, acc):
        cid = lax.axis_index("c")
        sid = lax.axis_index("s")
        w = cid * NS + sid
        pltpu.sync_copy(ones_in.at[0], ones_v)
        pltpu.sync_copy(ones_in.at[1], zero_v)
        _zero_acc(zero_v, acc, sid)
        pltpu.sync_copy(dstb.at[pl.ds(w * K, K)], idx_v)
        plsc.subcore_barrier()

        def step(j, _):
            pltpu.sync_copy(ones_v, acc.at[idx_v.at[j]], add=True)
            return 0

        lax.fori_loop(0, K, step, 0)
        plsc.subcore_barrier()
        _copy_out(acc, out, cid, sid)

    return deg_kernel


@functools.lru_cache(maxsize=None)
def _make_agg_kernel(hp):
    @functools.partial(
        pl.kernel,
        out_type=jax.ShapeDtypeStruct((2 * N, hp), jnp.float32),
        mesh=_mesh(),
        compiler_params=pltpu.CompilerParams(use_tc_tiling_on_sc=False),
        scratch_types=[
            pltpu.VMEM((max(KA[hp], 2 * K - KA[hp]), MB), jnp.int32),
            pltpu.VMEM((max(KA[hp], 2 * K - KA[hp]), MB), jnp.int32),
            pltpu.VMEM((MB, hp), jnp.float32),
            pltpu.VMEM_SHARED((ACCROWS, hp), jnp.float32),
            pltpu.SemaphoreType.DMA,
        ],
    )
    def agg_kernel(h, zblk, srcb, dstb, out, src_v, dst_v, rows_v, acc,
                   sem0):
        cid = lax.axis_index("c")
        sid = lax.axis_index("s")
        _agg_phase(h, zblk, out, srcb, dstb, src_v, dst_v, rows_v, acc, sem0,
                   cid, sid, hp)

    return agg_kernel


def _dinv_from(degp_ref):
    degc = degp_ref[0:N, 0:1] + degp_ref[N:2 * N, 0:1] + 1.0
    return lax.rsqrt(degc)


def _combine(p_ref, h_ref):
    return p_ref[0:N] + p_ref[N:2 * N] + h_ref[...]


def _t1_body(x_ref, w_ref, degp_ref, o_ref):
    dinv = _dinv_from(degp_ref)
    o_ref[...] = jnp.dot(x_ref[...], w_ref[...],
                         preferred_element_type=jnp.float32) * dinv


def _tmid_body(p_ref, hs_ref, degp_ref, b_ref, g_ref, be_ref, w_ref, o_ref):
    dinv = _dinv_from(degp_ref)
    a = _combine(p_ref, hs_ref) * dinv + b_ref[...]
    mean = jnp.mean(a, axis=0, keepdims=True)
    var = jnp.mean((a - mean) ** 2, axis=0, keepdims=True)
    y = (a - mean) * lax.rsqrt(var + 1e-5) * g_ref[...] + be_ref[...]
    y = jnp.maximum(y, 0.0)
    o_ref[...] = jnp.dot(y, w_ref[...],
                         preferred_element_type=jnp.float32) * dinv


def _t4_body(p_ref, hs_ref, degp_ref, b_ref, o_ref):
    dinv = _dinv_from(degp_ref)
    a = _combine(p_ref, hs_ref) * dinv + b_ref[...]
    o_ref[...] = a[:, 0:C]


def kernel(x, edge_index, W1, b1, g1, be1, W2, b2, g2, be2, W3, b3):
    pad = EPAD - E
    src = jnp.concatenate([edge_index[0], jnp.zeros((pad,), jnp.int32)])
    dst = jnp.concatenate([edge_index[1], jnp.full((pad,), N, jnp.int32)])
    srcb = src.reshape(KT, MB)
    dstb = dst.reshape(KT, MB)
    W3p = jnp.pad(W3, ((0, 0), (0, HH - C)))
    b3p = jnp.pad(b3, (0, HH - C)).reshape(1, HH)

    onesz = jnp.stack([jnp.ones((MB, DEGW), jnp.float32),
                       jnp.zeros((MB, DEGW), jnp.float32)])
    z128 = jnp.zeros((MB, H), jnp.float32)
    z48 = jnp.zeros((MB, HH), jnp.float32)
    degp = _make_deg_kernel()(dstb, onesz)
    agg_h = _make_agg_kernel(H)
    agg_c = _make_agg_kernel(HH)
    tc_params = pltpu.CompilerParams(vmem_limit_bytes=100 * 1024 * 1024)

    t1 = pl.pallas_call(
        _t1_body, compiler_params=tc_params,
        out_shape=jax.ShapeDtypeStruct((N, H), jnp.float32))
    h1 = t1(x, W1, degp)
    p1 = agg_h(h1, z128, srcb, dstb)

    tmid = pl.pallas_call(
        _tmid_body, compiler_params=tc_params,
        out_shape=jax.ShapeDtypeStruct((N, H), jnp.float32))
    h2 = tmid(p1, h1, degp, b1.reshape(1, H), g1.reshape(1, H),
              be1.reshape(1, H), W2)
    p2 = agg_h(h2, z128, srcb, dstb)

    t3 = pl.pallas_call(
        _tmid_body, compiler_params=tc_params,
        out_shape=jax.ShapeDtypeStruct((N, HH), jnp.float32))
    h3 = t3(p2, h2, degp, b2.reshape(1, H), g2.reshape(1, H),
            be2.reshape(1, H), W3p)
    p3 = agg_c(h3, z48, srcb, dstb)

    t4 = pl.pallas_call(
        _t4_body, compiler_params=tc_params,
        out_shape=jax.ShapeDtypeStruct((N, C), jnp.float32))
    return t4(p3, h3, degp, b3p)


# speedup vs baseline: 1.1303x; 1.1303x over previous
import functools

import jax
import jax.numpy as jnp
from jax import lax
from jax.experimental import pallas as pl
from jax.experimental.pallas import tpu as pltpu
from jax.experimental.pallas import tpu_sc as plsc

N = 10000
E = 320000
D = 128
H = 128
C = 40
HH = 64

NC = 2
NS = 16
NW = NC * NS
MB = 128
K = E // (NW * MB) + 1
KA = {128: 104, 64: 98}
KAMAX = max(KA.values())
KT = NS * 2 * K
ACCROWS = ((N + 1 + MB - 1) // MB) * MB
EPAD = KT * MB
RPT = 624
RPT_LAST = N - (NS - 1) * RPT
DEGW = 16


@functools.lru_cache(maxsize=None)
def _mesh():
    return plsc.VectorSubcoreMesh(core_axis_name="c", subcore_axis_name="s")


def _fill(ref, rows, cols, value):
    v16 = jnp.full((16,), value, jnp.float32)

    def row(i, _):
        def col(l, __):
            ref[i, pl.ds(l * 16, 16)] = v16
            return 0
        return lax.fori_loop(0, cols // 16, col, 0)

    lax.fori_loop(0, rows, row, 0)


def _zero_acc(zref, acc, sid):
    nchunks = ACCROWS // MB

    def step(t, _):
        cid = t * NS + sid

        @pl.when(cid < nchunks)
        def _():
            pltpu.sync_copy(zref, acc.at[pl.ds(cid * MB, MB)])
        return 0

    lax.fori_loop(0, (nchunks + NS - 1) // NS, step, 0)


def _copy_out(acc, out, cid, sid):
    @pl.when(sid < NS - 1)
    def _():
        pltpu.sync_copy(acc.at[pl.ds(sid * RPT, RPT)],
                        out.at[pl.ds(cid * N + sid * RPT, RPT)])

    @pl.when(sid == NS - 1)
    def _():
        pltpu.sync_copy(acc.at[pl.ds((NS - 1) * RPT, RPT_LAST)],
                        out.at[pl.ds(cid * N + (NS - 1) * RPT, RPT_LAST)])


def _agg_phase(h, out, srcb, dstb, src_v, dst_v, rows_v, acc, sem0,
               cid, sid, hp):
    ka = KA[hp]
    kb = 2 * K - ka
    plsc.subcore_barrier()
    _fill(rows_v, MB, hp, 0.0)
    _zero_acc(rows_v, acc, sid)
    plsc.subcore_barrier()

    def work(kc, start):
        pltpu.sync_copy(srcb.at[pl.ds(start, kc)], src_v.at[pl.ds(0, kc)])
        pltpu.sync_copy(dstb.at[pl.ds(start, kc)], dst_v.at[pl.ds(0, kc)])

        def step(g, _):
            pltpu.async_copy(h.at[src_v.at[g]], rows_v, sem0).wait()
            pltpu.sync_copy(rows_v, acc.at[dst_v.at[g]], add=True)
            return 0

        lax.fori_loop(0, kc, step, 0)

    @pl.when(cid == 0)
    def _():
        work(ka, sid * ka)

    @pl.when(cid == 1)
    def _():
        work(kb, NS * ka + sid * kb)

    plsc.subcore_barrier()
    _copy_out(acc, out, cid, sid)


@functools.lru_cache(maxsize=None)
def _make_deg_kernel():
    @functools.partial(
        pl.kernel,
        out_type=jax.ShapeDtypeStruct((2 * N, DEGW), jnp.float32),
        mesh=_mesh(),
        compiler_params=pltpu.CompilerParams(use_tc_tiling_on_sc=False),
        scratch_types=[
            pltpu.VMEM((K, MB), jnp.int32),
            pltpu.VMEM((MB, DEGW), jnp.float32),
            pltpu.VMEM((MB, DEGW), jnp.float32),
            pltpu.VMEM_SHARED((ACCROWS, DEGW), jnp.float32),
        ],
    )
    def deg_kernel(dstb, out, idx_v, ones_v, zero_v, acc):
        cid = lax.axis_index("c")
        sid = lax.axis_index("s")
        w = cid * NS + sid
        _fill(ones_v, MB, DEGW, 1.0)
        _fill(zero_v, MB, DEGW, 0.0)
        _zero_acc(zero_v, acc, sid)
        pltpu.sync_copy(dstb.at[pl.ds(w * K, K)], idx_v)
        plsc.subcore_barrier()

        def step(j, _):
            pltpu.sync_copy(ones_v, acc.at[idx_v.at[j]], add=True)
            return 0

        lax.fori_loop(0, K, step, 0)
        plsc.subcore_barrier()
        _copy_out(acc, out, cid, sid)

    return deg_kernel


@functools.lru_cache(maxsize=None)
def _make_agg_kernel(hp):
    @functools.partial(
        pl.kernel,
        out_type=jax.ShapeDtypeStruct((2 * N, hp), jnp.float32),
        mesh=_mesh(),
        compiler_params=pltpu.CompilerParams(use_tc_tiling_on_sc=False),
        scratch_types=[
            pltpu.VMEM((max(KA[hp], 2 * K - KA[hp]), MB), jnp.int32),
            pltpu.VMEM((max(KA[hp], 2 * K - KA[hp]), MB), jnp.int32),
            pltpu.VMEM((MB, hp), jnp.float32),
            pltpu.VMEM_SHARED((ACCROWS, hp), jnp.float32),
            pltpu.SemaphoreType.DMA,
        ],
    )
    def agg_kernel(h, srcb, dstb, out, src_v, dst_v, rows_v, acc, sem0):
        cid = lax.axis_index("c")
        sid = lax.axis_index("s")
        _agg_phase(h, out, srcb, dstb, src_v, dst_v, rows_v, acc, sem0,
                   cid, sid, hp)

    return agg_kernel


def _dinv_from(degp_ref):
    degc = degp_ref[0:N, 0:1] + degp_ref[N:2 * N, 0:1] + 1.0
    return lax.rsqrt(degc)


def _combine(p_ref, h_ref):
    return p_ref[0:N] + p_ref[N:2 * N] + h_ref[...]


def _t1_body(x_ref, w_ref, degp_ref, o_ref):
    dinv = _dinv_from(degp_ref)
    o_ref[...] = jnp.dot(x_ref[...], w_ref[...],
                         preferred_element_type=jnp.float32) * dinv


def _tmid_body(p_ref, hs_ref, degp_ref, b_ref, g_ref, be_ref, w_ref, o_ref):
    dinv = _dinv_from(degp_ref)
    a = _combine(p_ref, hs_ref) * dinv + b_ref[...]
    mean = jnp.mean(a, axis=0, keepdims=True)
    var = jnp.mean((a - mean) ** 2, axis=0, keepdims=True)
    y = (a - mean) * lax.rsqrt(var + 1e-5) * g_ref[...] + be_ref[...]
    y = jnp.maximum(y, 0.0)
    o_ref[...] = jnp.dot(y, w_ref[...],
                         preferred_element_type=jnp.float32) * dinv


def _t4_body(p_ref, hs_ref, degp_ref, b_ref, o_ref):
    dinv = _dinv_from(degp_ref)
    a = _combine(p_ref, hs_ref) * dinv + b_ref[...]
    o_ref[...] = a[:, 0:C]


def kernel(x, edge_index, W1, b1, g1, be1, W2, b2, g2, be2, W3, b3):
    pad = EPAD - E
    src = jnp.concatenate([edge_index[0], jnp.zeros((pad,), jnp.int32)])
    dst = jnp.concatenate([edge_index[1], jnp.full((pad,), N, jnp.int32)])
    srcb = src.reshape(KT, MB)
    dstb = dst.reshape(KT, MB)
    W3p = jnp.pad(W3, ((0, 0), (0, HH - C)))
    b3p = jnp.pad(b3, (0, HH - C)).reshape(1, HH)

    degp = _make_deg_kernel()(dstb)
    agg_h = _make_agg_kernel(H)
    agg_c = _make_agg_kernel(HH)
    tc_params = pltpu.CompilerParams(vmem_limit_bytes=100 * 1024 * 1024)

    t1 = pl.pallas_call(
        _t1_body, compiler_params=tc_params,
        out_shape=jax.ShapeDtypeStruct((N, H), jnp.float32))
    h1 = t1(x, W1, degp)
    p1 = agg_h(h1, srcb, dstb)

    tmid = pl.pallas_call(
        _tmid_body, compiler_params=tc_params,
        out_shape=jax.ShapeDtypeStruct((N, H), jnp.float32))
    h2 = tmid(p1, h1, degp, b1.reshape(1, H), g1.reshape(1, H),
              be1.reshape(1, H), W2)
    p2 = agg_h(h2, srcb, dstb)

    t3 = pl.pallas_call(
        _tmid_body, compiler_params=tc_params,
        out_shape=jax.ShapeDtypeStruct((N, HH), jnp.float32))
    h3 = t3(p2, h2, degp, b2.reshape(1, H), g2.reshape(1, H),
            be2.reshape(1, H), W3p)
    p3 = agg_c(h3, srcb, dstb)

    t4 = pl.pallas_call(
        _t4_body, compiler_params=tc_params,
        out_shape=jax.ShapeDtypeStruct((N, C), jnp.float32))
    return t4(p3, h3, degp, b3p)


# speedup vs baseline: 1.1412x; 1.0096x over previous
import functools

import jax
import jax.numpy as jnp
from jax import lax
from jax.experimental import pallas as pl
from jax.experimental.pallas import tpu as pltpu
from jax.experimental.pallas import tpu_sc as plsc

N = 10000
E = 320000
D = 128
H = 128
C = 40
HH = 64

NC = 2
NS = 16
NW = NC * NS
MB = 128
K = E // (NW * MB) + 1
KA = {128: 108, 64: 102}
KAMAX = max(KA.values())
KT = NS * 2 * K
ACCROWS = ((N + 1 + MB - 1) // MB) * MB
EPAD = KT * MB
RPT = 624
RPT_LAST = N - (NS - 1) * RPT
DEGW = 16


@functools.lru_cache(maxsize=None)
def _mesh():
    return plsc.VectorSubcoreMesh(core_axis_name="c", subcore_axis_name="s")


def _fill(ref, rows, cols, value):
    v16 = jnp.full((16,), value, jnp.float32)

    def row(i, _):
        def col(l, __):
            ref[i, pl.ds(l * 16, 16)] = v16
            return 0
        return lax.fori_loop(0, cols // 16, col, 0)

    lax.fori_loop(0, rows, row, 0)


def _zero_acc(zref, acc, sid):
    nchunks = ACCROWS // MB

    def step(t, _):
        cid = t * NS + sid

        @pl.when(cid < nchunks)
        def _():
            pltpu.sync_copy(zref, acc.at[pl.ds(cid * MB, MB)])
        return 0

    lax.fori_loop(0, (nchunks + NS - 1) // NS, step, 0)


def _copy_out(acc, out, cid, sid):
    @pl.when(sid < NS - 1)
    def _():
        pltpu.sync_copy(acc.at[pl.ds(sid * RPT, RPT)],
                        out.at[pl.ds(cid * N + sid * RPT, RPT)])

    @pl.when(sid == NS - 1)
    def _():
        pltpu.sync_copy(acc.at[pl.ds((NS - 1) * RPT, RPT_LAST)],
                        out.at[pl.ds(cid * N + (NS - 1) * RPT, RPT_LAST)])


def _agg_phase(h, out, srcb, dstb, src_v, dst_v, rows_v, acc, sem0,
               cid, sid, hp):
    ka = KA[hp]
    kb = 2 * K - ka
    plsc.subcore_barrier()
    _fill(rows_v, MB, hp, 0.0)
    _zero_acc(rows_v, acc, sid)
    plsc.subcore_barrier()

    def work(kc, start):
        pltpu.sync_copy(srcb.at[pl.ds(start, kc)], src_v.at[pl.ds(0, kc)])
        pltpu.sync_copy(dstb.at[pl.ds(start, kc)], dst_v.at[pl.ds(0, kc)])

        def step(g, _):
            pltpu.async_copy(h.at[src_v.at[g]], rows_v, sem0).wait()
            pltpu.sync_copy(rows_v, acc.at[dst_v.at[g]], add=True)
            return 0

        lax.fori_loop(0, kc, step, 0)

    @pl.when(cid == 0)
    def _():
        work(ka, sid * ka)

    @pl.when(cid == 1)
    def _():
        work(kb, NS * ka + sid * kb)

    plsc.subcore_barrier()
    _copy_out(acc, out, cid, sid)


@functools.lru_cache(maxsize=None)
def _make_deg_kernel():
    @functools.partial(
        pl.kernel,
        out_type=jax.ShapeDtypeStruct((2 * N, DEGW), jnp.float32),
        mesh=_mesh(),
        compiler_params=pltpu.CompilerParams(use_tc_tiling_on_sc=False),
        scratch_types=[
            pltpu.VMEM((K, MB), jnp.int32),
            pltpu.VMEM((MB, DEGW), jnp.float32),
            pltpu.VMEM((MB, DEGW), jnp.float32),
            pltpu.VMEM_SHARED((ACCROWS, DEGW), jnp.float32),
        ],
    )
    def deg_kernel(dstb, out, idx_v, ones_v, zero_v, acc):
        cid = lax.axis_index("c")
        sid = lax.axis_index("s")
        w = cid * NS + sid
        _fill(ones_v, MB, DEGW, 1.0)
        _fill(zero_v, MB, DEGW, 0.0)
        _zero_acc(zero_v, acc, sid)
        pltpu.sync_copy(dstb.at[pl.ds(w * K, K)], idx_v)
        plsc.subcore_barrier()

        def step(j, _):
            pltpu.sync_copy(ones_v, acc.at[idx_v.at[j]], add=True)
            return 0

        lax.fori_loop(0, K, step, 0)
        plsc.subcore_barrier()
        _copy_out(acc, out, cid, sid)

    return deg_kernel


@functools.lru_cache(maxsize=None)
def _make_agg_kernel(hp):
    @functools.partial(
        pl.kernel,
        out_type=jax.ShapeDtypeStruct((2 * N, hp), jnp.float32),
        mesh=_mesh(),
        compiler_params=pltpu.CompilerParams(use_tc_tiling_on_sc=False),
        scratch_types=[
            pltpu.VMEM((max(KA[hp], 2 * K - KA[hp]), MB), jnp.int32),
            pltpu.VMEM((max(KA[hp], 2 * K - KA[hp]), MB), jnp.int32),
            pltpu.VMEM((MB, hp), jnp.float32),
            pltpu.VMEM_SHARED((ACCROWS, hp), jnp.float32),
            pltpu.SemaphoreType.DMA,
        ],
    )
    def agg_kernel(h, srcb, dstb, out, src_v, dst_v, rows_v, acc, sem0):
        cid = lax.axis_index("c")
        sid = lax.axis_index("s")
        _agg_phase(h, out, srcb, dstb, src_v, dst_v, rows_v, acc, sem0,
                   cid, sid, hp)

    return agg_kernel


def _dinv_from(degp_ref):
    degc = degp_ref[0:N, 0:1] + degp_ref[N:2 * N, 0:1] + 1.0
    return lax.rsqrt(degc)


def _combine(p_ref, h_ref):
    return p_ref[0:N] + p_ref[N:2 * N] + h_ref[...]


def _t1_body(x_ref, w_ref, degp_ref, o_ref):
    dinv = _dinv_from(degp_ref)
    o_ref[...] = jnp.dot(x_ref[...], w_ref[...],
                         preferred_element_type=jnp.float32) * dinv


def _tmid_body(p_ref, hs_ref, degp_ref, b_ref, g_ref, be_ref, w_ref, o_ref):
    dinv = _dinv_from(degp_ref)
    a = _combine(p_ref, hs_ref) * dinv + b_ref[...]
    mean = jnp.mean(a, axis=0, keepdims=True)
    var = jnp.mean((a - mean) ** 2, axis=0, keepdims=True)
    y = (a - mean) * lax.rsqrt(var + 1e-5) * g_ref[...] + be_ref[...]
    y = jnp.maximum(y, 0.0)
    o_ref[...] = jnp.dot(y, w_ref[...],
                         preferred_element_type=jnp.float32) * dinv


def _t4_body(p_ref, hs_ref, degp_ref, b_ref, o_ref):
    dinv = _dinv_from(degp_ref)
    a = _combine(p_ref, hs_ref) * dinv + b_ref[...]
    o_ref[...] = a[:, 0:C]


def kernel(x, edge_index, W1, b1, g1, be1, W2, b2, g2, be2, W3, b3):
    pad = EPAD - E
    src = jnp.concatenate([edge_index[0], jnp.zeros((pad,), jnp.int32)])
    dst = jnp.concatenate([edge_index[1], jnp.full((pad,), N, jnp.int32)])
    srcb = src.reshape(KT, MB)
    dstb = dst.reshape(KT, MB)
    W3p = jnp.pad(W3, ((0, 0), (0, HH - C)))
    b3p = jnp.pad(b3, (0, HH - C)).reshape(1, HH)

    degp = _make_deg_kernel()(dstb)
    agg_h = _make_agg_kernel(H)
    agg_c = _make_agg_kernel(HH)
    tc_params = pltpu.CompilerParams(vmem_limit_bytes=100 * 1024 * 1024)

    t1 = pl.pallas_call(
        _t1_body, compiler_params=tc_params,
        out_shape=jax.ShapeDtypeStruct((N, H), jnp.float32))
    h1 = t1(x, W1, degp)
    p1 = agg_h(h1, srcb, dstb)

    tmid = pl.pallas_call(
        _tmid_body, compiler_params=tc_params,
        out_shape=jax.ShapeDtypeStruct((N, H), jnp.float32))
    h2 = tmid(p1, h1, degp, b1.reshape(1, H), g1.reshape(1, H),
              be1.reshape(1, H), W2)
    p2 = agg_h(h2, srcb, dstb)

    t3 = pl.pallas_call(
        _tmid_body, compiler_params=tc_params,
        out_shape=jax.ShapeDtypeStruct((N, HH), jnp.float32))
    h3 = t3(p2, h2, degp, b2.reshape(1, H), g2.reshape(1, H),
            be2.reshape(1, H), W3p)
    p3 = agg_c(h3, srcb, dstb)

    t4 = pl.pallas_call(
        _t4_body, compiler_params=tc_params,
        out_shape=jax.ShapeDtypeStruct((N, C), jnp.float32))
    return t4(p3, h3, degp, b3p)


# speedup vs baseline: 1.1464x; 1.0046x over previous
import functools

import jax
import jax.numpy as jnp
from jax import lax
from jax.experimental import pallas as pl
from jax.experimental.pallas import tpu as pltpu
from jax.experimental.pallas import tpu_sc as plsc

N = 10000
E = 320000
D = 128
H = 128
C = 40
HH = 64

NC = 2
NS = 16
NW = NC * NS
MB = 128
K = E // (NW * MB) + 1
KA = {128: 112, 64: 106}
KAMAX = max(KA.values())
KT = NS * 2 * K
ACCROWS = ((N + 1 + MB - 1) // MB) * MB
EPAD = KT * MB
RPT = 624
RPT_LAST = N - (NS - 1) * RPT
DEGW = 16


@functools.lru_cache(maxsize=None)
def _mesh():
    return plsc.VectorSubcoreMesh(core_axis_name="c", subcore_axis_name="s")


def _fill(ref, rows, cols, value):
    v16 = jnp.full((16,), value, jnp.float32)

    def row(i, _):
        def col(l, __):
            ref[i, pl.ds(l * 16, 16)] = v16
            return 0
        return lax.fori_loop(0, cols // 16, col, 0)

    lax.fori_loop(0, rows, row, 0)


def _zero_acc(zref, acc, sid):
    nchunks = ACCROWS // MB

    def step(t, _):
        cid = t * NS + sid

        @pl.when(cid < nchunks)
        def _():
            pltpu.sync_copy(zref, acc.at[pl.ds(cid * MB, MB)])
        return 0

    lax.fori_loop(0, (nchunks + NS - 1) // NS, step, 0)


def _copy_out(acc, out, cid, sid):
    @pl.when(sid < NS - 1)
    def _():
        pltpu.sync_copy(acc.at[pl.ds(sid * RPT, RPT)],
                        out.at[pl.ds(cid * N + sid * RPT, RPT)])

    @pl.when(sid == NS - 1)
    def _():
        pltpu.sync_copy(acc.at[pl.ds((NS - 1) * RPT, RPT_LAST)],
                        out.at[pl.ds(cid * N + (NS - 1) * RPT, RPT_LAST)])


def _agg_phase(h, out, srcb, dstb, src_v, dst_v, rows_v, acc, sem0,
               cid, sid, hp):
    ka = KA[hp]
    kb = 2 * K - ka
    plsc.subcore_barrier()
    _fill(rows_v, MB, hp, 0.0)
    _zero_acc(rows_v, acc, sid)
    plsc.subcore_barrier()

    def work(kc, start):
        pltpu.sync_copy(srcb.at[pl.ds(start, kc)], src_v.at[pl.ds(0, kc)])
        pltpu.sync_copy(dstb.at[pl.ds(start, kc)], dst_v.at[pl.ds(0, kc)])

        def step(g, _):
            pltpu.async_copy(h.at[src_v.at[g]], rows_v, sem0).wait()
            pltpu.sync_copy(rows_v, acc.at[dst_v.at[g]], add=True)
            return 0

        lax.fori_loop(0, kc, step, 0)

    @pl.when(cid == 0)
    def _():
        work(ka, sid * ka)

    @pl.when(cid == 1)
    def _():
        work(kb, NS * ka + sid * kb)

    plsc.subcore_barrier()
    _copy_out(acc, out, cid, sid)


@functools.lru_cache(maxsize=None)
def _make_deg_kernel():
    @functools.partial(
        pl.kernel,
        out_type=jax.ShapeDtypeStruct((2 * N, DEGW), jnp.float32),
        mesh=_mesh(),
        compiler_params=pltpu.CompilerParams(use_tc_tiling_on_sc=False),
        scratch_types=[
            pltpu.VMEM((K, MB), jnp.int32),
            pltpu.VMEM((MB, DEGW), jnp.float32),
            pltpu.VMEM((MB, DEGW), jnp.float32),
            pltpu.VMEM_SHARED((ACCROWS, DEGW), jnp.float32),
        ],
    )
    def deg_kernel(dstb, out, idx_v, ones_v, zero_v, acc):
        cid = lax.axis_index("c")
        sid = lax.axis_index("s")
        w = cid * NS + sid
        _fill(ones_v, MB, DEGW, 1.0)
        _fill(zero_v, MB, DEGW, 0.0)
        _zero_acc(zero_v, acc, sid)
        pltpu.sync_copy(dstb.at[pl.ds(w * K, K)], idx_v)
        plsc.subcore_barrier()

        def step(j, _):
            pltpu.sync_copy(ones_v, acc.at[idx_v.at[j]], add=True)
            return 0

        lax.fori_loop(0, K, step, 0)
        plsc.subcore_barrier()
        _copy_out(acc, out, cid, sid)

    return deg_kernel


@functools.lru_cache(maxsize=None)
def _make_agg_kernel(hp):
    @functools.partial(
        pl.kernel,
        out_type=jax.ShapeDtypeStruct((2 * N, hp), jnp.float32),
        mesh=_mesh(),
        compiler_params=pltpu.CompilerParams(use_tc_tiling_on_sc=False),
        scratch_types=[
            pltpu.VMEM((max(KA[hp], 2 * K - KA[hp]), MB), jnp.int32),
            pltpu.VMEM((max(KA[hp], 2 * K - KA[hp]), MB), jnp.int32),
            pltpu.VMEM((MB, hp), jnp.float32),
            pltpu.VMEM_SHARED((ACCROWS, hp), jnp.float32),
            pltpu.SemaphoreType.DMA,
        ],
    )
    def agg_kernel(h, srcb, dstb, out, src_v, dst_v, rows_v, acc, sem0):
        cid = lax.axis_index("c")
        sid = lax.axis_index("s")
        _agg_phase(h, out, srcb, dstb, src_v, dst_v, rows_v, acc, sem0,
                   cid, sid, hp)

    return agg_kernel


def _dinv_from(degp_ref):
    degc = degp_ref[0:N, 0:1] + degp_ref[N:2 * N, 0:1] + 1.0
    return lax.rsqrt(degc)


def _combine(p_ref, h_ref):
    return p_ref[0:N] + p_ref[N:2 * N] + h_ref[...]


def _t1_body(x_ref, w_ref, degp_ref, o_ref):
    dinv = _dinv_from(degp_ref)
    o_ref[...] = jnp.dot(x_ref[...], w_ref[...],
                         preferred_element_type=jnp.float32) * dinv


def _tmid_body(p_ref, hs_ref, degp_ref, b_ref, g_ref, be_ref, w_ref, o_ref):
    dinv = _dinv_from(degp_ref)
    a = _combine(p_ref, hs_ref) * dinv + b_ref[...]
    mean = jnp.mean(a, axis=0, keepdims=True)
    var = jnp.mean((a - mean) ** 2, axis=0, keepdims=True)
    y = (a - mean) * lax.rsqrt(var + 1e-5) * g_ref[...] + be_ref[...]
    y = jnp.maximum(y, 0.0)
    o_ref[...] = jnp.dot(y, w_ref[...],
                         preferred_element_type=jnp.float32) * dinv


def _t4_body(p_ref, hs_ref, degp_ref, b_ref, o_ref):
    dinv = _dinv_from(degp_ref)
    a = _combine(p_ref, hs_ref) * dinv + b_ref[...]
    o_ref[...] = a[:, 0:C]


def kernel(x, edge_index, W1, b1, g1, be1, W2, b2, g2, be2, W3, b3):
    pad = EPAD - E
    src = jnp.concatenate([edge_index[0], jnp.zeros((pad,), jnp.int32)])
    dst = jnp.concatenate([edge_index[1], jnp.full((pad,), N, jnp.int32)])
    srcb = src.reshape(KT, MB)
    dstb = dst.reshape(KT, MB)
    W3p = jnp.pad(W3, ((0, 0), (0, HH - C)))
    b3p = jnp.pad(b3, (0, HH - C)).reshape(1, HH)

    degp = _make_deg_kernel()(dstb)
    agg_h = _make_agg_kernel(H)
    agg_c = _make_agg_kernel(HH)
    tc_params = pltpu.CompilerParams(vmem_limit_bytes=100 * 1024 * 1024)

    t1 = pl.pallas_call(
        _t1_body, compiler_params=tc_params,
        out_shape=jax.ShapeDtypeStruct((N, H), jnp.float32))
    h1 = t1(x, W1, degp)
    p1 = agg_h(h1, srcb, dstb)

    tmid = pl.pallas_call(
        _tmid_body, compiler_params=tc_params,
        out_shape=jax.ShapeDtypeStruct((N, H), jnp.float32))
    h2 = tmid(p1, h1, degp, b1.reshape(1, H), g1.reshape(1, H),
              be1.reshape(1, H), W2)
    p2 = agg_h(h2, srcb, dstb)

    t3 = pl.pallas_call(
        _tmid_body, compiler_params=tc_params,
        out_shape=jax.ShapeDtypeStruct((N, HH), jnp.float32))
    h3 = t3(p2, h2, degp, b2.reshape(1, H), g2.reshape(1, H),
            be2.reshape(1, H), W3p)
    p3 = agg_c(h3, srcb, dstb)

    t4 = pl.pallas_call(
        _t4_body, compiler_params=tc_params,
        out_shape=jax.ShapeDtypeStruct((N, C), jnp.float32))
    return t4(p3, h3, degp, b3p)
